# baseline (speedup 1.0000x reference)
import functools

import jax
import jax.numpy as jnp
from jax import lax
from jax.experimental import pallas as pl
from jax.experimental.pallas import tpu as pltpu
from jax.experimental.pallas import tpu_sc as plsc

N, E, D, H, C = 10000, 320000, 128, 32, 40
CP = 64

NC, NS = 2, 16
NTILES = NC * NS
EPT = E // NTILES
K = 125
NCHUNK = EPT // K
EK = E // K
NP = 10240
STRIPE = NP // NS
ZROWS = STRIPE // 8

GRID = 1
PB4 = NP // 4 // GRID
PB2 = NP // 2 // GRID

_mesh = plsc.VectorSubcoreMesh(core_axis_name="c", subcore_axis_name="s")


@functools.partial(
    pl.kernel,
    out_type=jax.ShapeDtypeStruct((NC, NP, 16), jnp.float32),
    mesh=_mesh,
    compiler_params=pltpu.CompilerParams(use_tc_tiling_on_sc=False),
    scratch_types=[
        pltpu.VMEM((NCHUNK, K), jnp.int32),
        pltpu.VMEM((K, 16), jnp.float32),
        pltpu.VMEM((ZROWS, 16), jnp.float32),
        pltpu.VMEM_SHARED((NP, 16), jnp.float32),
        pltpu.SemaphoreType.DMA,
    ],
)
def _sc_degree(eidx_hbm, out_hbm, didx, ones, zbuf, acc, dsem):
    cid = lax.axis_index("c")
    sid = lax.axis_index("s")
    tile = cid * NS + sid

    ci = pltpu.async_copy(
        eidx_hbm.at[1, pl.ds(tile * NCHUNK, NCHUNK)], didx, dsem)

    @pl.loop(0, ZROWS)
    def _(i):
        zbuf[i, :] = jnp.zeros((16,), jnp.float32)

    @pl.loop(0, 8)
    def _(r):
        pltpu.sync_copy(zbuf, acc.at[pl.ds(sid * STRIPE + r * ZROWS, ZROWS)])

    @pl.loop(0, K)
    def _(i):
        ones[i, :] = jnp.ones((16,), jnp.float32)

    ci.wait()
    plsc.subcore_barrier()

    @pl.loop(0, 8)
    def _(j):
        pltpu.async_copy(ones, acc.at[didx.at[j]], dsem, add=True)

    @pl.loop(8, NCHUNK)
    def _(j):
        pltpu.make_async_copy(ones, acc.at[didx.at[0]], dsem).wait()
        pltpu.async_copy(ones, acc.at[didx.at[j]], dsem, add=True)

    @pl.loop(0, 8)
    def _(j):
        pltpu.make_async_copy(ones, acc.at[didx.at[0]], dsem).wait()

    plsc.subcore_barrier()
    pltpu.sync_copy(acc.at[pl.ds(sid * STRIPE, STRIPE)],
                    out_hbm.at[cid, pl.ds(sid * STRIPE, STRIPE)])


def _make_sc_aggregate(width):
    nbuf = 8
    @functools.partial(
        pl.kernel,
        out_type=jax.ShapeDtypeStruct((NC, NP, width), jnp.float32),
        mesh=_mesh,
        compiler_params=pltpu.CompilerParams(use_tc_tiling_on_sc=False),
        scratch_types=(
            [pltpu.VMEM((NCHUNK, K), jnp.int32)] * 2
            + [pltpu.VMEM((K, width), jnp.float32)] * nbuf
            + [pltpu.VMEM((ZROWS, width), jnp.float32),
               pltpu.VMEM_SHARED((NP, width), jnp.float32)]
            + [pltpu.SemaphoreType.DMA] * (2 * nbuf + 1)
        ),
    )
    def agg(rows_hbm, eidx_hbm, out_hbm, *scr):
        sidx, didx = scr[0], scr[1]
        bufs = scr[2:2 + nbuf]
        zbuf, acc = scr[2 + nbuf], scr[3 + nbuf]
        gsem = scr[4 + nbuf:4 + 2 * nbuf]
        ssem = scr[4 + 2 * nbuf:4 + 3 * nbuf]
        isem = scr[4 + 3 * nbuf]
        cid = lax.axis_index("c")
        sid = lax.axis_index("s")
        tile = cid * NS + sid

        ca = pltpu.async_copy(
            eidx_hbm.at[0, pl.ds(tile * NCHUNK, NCHUNK)], sidx, isem)
        cb = pltpu.async_copy(
            eidx_hbm.at[1, pl.ds(tile * NCHUNK, NCHUNK)], didx, isem)

        @pl.loop(0, ZROWS)
        def _(i):
            @pl.loop(0, width // 16)
            def _(cc):
                zbuf[i, pl.ds(cc * 16, 16)] = jnp.zeros((16,), jnp.float32)

        @pl.loop(0, 8)
        def _(r):
            pltpu.sync_copy(
                zbuf, acc.at[pl.ds(sid * STRIPE + r * ZROWS, ZROWS)])

        ca.wait()
        cb.wait()

        for u in range(nbuf - 1):
            pltpu.async_copy(rows_hbm.at[sidx.at[u]], bufs[u], gsem[u])

        plsc.subcore_barrier()

        @pl.loop(0, NCHUNK, step=nbuf)
        def _(j):
            for u in range(nbuf):
                c = j + u
                pn = (u + nbuf - 1) % nbuf
                pltpu.make_async_copy(
                    rows_hbm.at[sidx.at[c]], bufs[u], gsem[u]).wait()
                pltpu.async_copy(bufs[u], acc.at[didx.at[c]], ssem[u],
                                 add=True)

                @pl.when(c >= 1)
                def _():
                    pltpu.make_async_copy(
                        bufs[pn], acc.at[didx.at[c - 1]], ssem[pn]).wait()

                @pl.when(c + nbuf - 1 < NCHUNK)
                def _():
                    pltpu.async_copy(
                        rows_hbm.at[sidx.at[c + nbuf - 1]], bufs[pn],
                        gsem[pn])

        pltpu.make_async_copy(
            bufs[(NCHUNK - 1) % nbuf], acc.at[didx.at[NCHUNK - 1]],
            ssem[(NCHUNK - 1) % nbuf]).wait()
        plsc.subcore_barrier()
        pltpu.sync_copy(acc.at[pl.ds(sid * STRIPE, STRIPE)],
                        out_hbm.at[cid, pl.ds(sid * STRIPE, STRIPE)])

    return agg


_sc_agg_h = _make_sc_aggregate(H)


def _lane_shift(w, off, total=128):
    parts = []
    if off:
        parts.append(jnp.zeros((w.shape[0], off), w.dtype))
    parts.append(w)
    rem = total - off - w.shape[1]
    if rem:
        parts.append(jnp.zeros((w.shape[0], rem), w.dtype))
    return jnp.concatenate(parts, axis=1) if len(parts) > 1 else w


def _blockdiag4(w):
    return jnp.concatenate([_lane_shift(w, 32 * a) for a in range(4)], axis=0)


def _expand16(x):
    return jnp.concatenate(
        [x[:, 16 * g:16 * g + 16] for g in (0, 0, 1, 1, 2, 2, 3, 3)], axis=1)


def _dinv4(degp_ref):
    d16 = lax.rsqrt(degp_ref[0] + degp_ref[1] + 1.0)
    return _interleave(_expand16(d16[:, :64]), _expand16(d16[:, 64:]))


def _tc_first_body(xp_ref, glove_ref, w0_ref, degp_ref, out_ref):
    w0p = jnp.dot(glove_ref[...], w0_ref[...],
                  preferred_element_type=jnp.float32)
    wblk = jnp.concatenate(
        [_lane_shift(w0p, 32 * a) for a in range(4)], axis=0)
    h = jnp.dot(xp_ref[...], wblk, preferred_element_type=jnp.float32)
    out_ref[...] = h * _dinv4(degp_ref)


def _tc_first(xp, glove, W0, degp):
    return pl.pallas_call(
        _tc_first_body,
        grid=(GRID,),
        in_specs=[
            pl.BlockSpec((PB4, 4 * D), lambda i: (i, 0)),
            pl.BlockSpec((D, D), lambda i: (0, 0)),
            pl.BlockSpec((D, H), lambda i: (0, 0)),
            pl.BlockSpec((NC, PB4 // 2, 128), lambda i: (0, i, 0)),
        ],
        out_specs=pl.BlockSpec((PB4, 128), lambda i: (i, 0)),
        out_shape=jax.ShapeDtypeStruct((NP // 4, 128), jnp.float32),
    )(xp, glove, W0, degp)


def _pre_act(p_ref, h_ref, b_ref, dinv):
    bt = jnp.concatenate([b_ref[...]] * 4, axis=1)
    pre = dinv * (p_ref[0] + p_ref[1] + h_ref[...]) + bt
    return jnp.maximum(pre, 0.0)


def _tc_mid_body(p_ref, h_ref, degp_ref, b_ref, w_ref, out_ref):
    dinv = _dinv4(degp_ref)
    act = _pre_act(p_ref, h_ref, b_ref, dinv)
    wblk = _blockdiag4(w_ref[...])
    out_ref[...] = jnp.dot(
        act, wblk, preferred_element_type=jnp.float32) * dinv


def _tc_mid(p, hprev, degp, b, w):
    return pl.pallas_call(
        _tc_mid_body,
        grid=(GRID,),
        in_specs=[
            pl.BlockSpec((NC, PB4, 128), lambda i: (0, i, 0)),
            pl.BlockSpec((PB4, 128), lambda i: (i, 0)),
            pl.BlockSpec((NC, PB4 // 2, 128), lambda i: (0, i, 0)),
            pl.BlockSpec((1, H), lambda i: (0, 0)),
            pl.BlockSpec((H, H), lambda i: (0, 0)),
        ],
        out_specs=pl.BlockSpec((PB4, 128), lambda i: (i, 0)),
        out_shape=jax.ShapeDtypeStruct((NP // 4, 128), jnp.float32),
    )(p, hprev, degp, b, w)


def _interleave(even, odd):
    r = even.shape[0]
    st = jnp.concatenate([jnp.reshape(even, (r, 1, 128)),
                          jnp.reshape(odd, (r, 1, 128))], axis=1)
    return jnp.reshape(st, (2 * r, 128))


def _regroup64(d4, lo):
    s = 0 if lo else 64
    a = d4[:, s:s + 32]
    b = d4[:, s + 32:s + 64]
    return jnp.concatenate([a, a, b, b], axis=1)


def _tc_mid2_body(p_ref, h_ref, degp_ref, b_ref, out_ref):
    dinv = _dinv4(degp_ref)
    act = _pre_act(p_ref, h_ref, b_ref, dinv)
    out_ref[...] = act * dinv


def _tc_mid2(p, hprev, degp, b):
    return pl.pallas_call(
        _tc_mid2_body,
        grid=(GRID,),
        in_specs=[
            pl.BlockSpec((NC, PB4, 128), lambda i: (0, i, 0)),
            pl.BlockSpec((PB4, 128), lambda i: (i, 0)),
            pl.BlockSpec((NC, PB4 // 2, 128), lambda i: (0, i, 0)),
            pl.BlockSpec((1, H), lambda i: (0, 0)),
        ],
        out_specs=pl.BlockSpec((PB4, 128), lambda i: (i, 0)),
        out_shape=jax.ShapeDtypeStruct((NP // 4, 128), jnp.float32),
    )(p, hprev, degp, b)


def _tc_final_body(p_ref, g_ref, degp_ref, b_ref, w_ref, out_ref):
    dinv = _dinv4(degp_ref)
    sagg = p_ref[0] + p_ref[1] + g_ref[...]
    w2e = _lane_shift(w_ref[...], 0, 128)
    bt = _lane_shift(b_ref[...], 0, 128)
    rows = []
    for a in range(4):
        sa = sagg[:, 32 * a:32 * a + 32]
        va = jnp.dot(sa, w2e, preferred_element_type=jnp.float32)
        da = dinv[:, 32 * a:32 * a + 1]
        rows.append(jnp.reshape(va * da + bt, (PB4, 1, 128)))
    pre = jnp.reshape(jnp.concatenate(rows, axis=1), (4 * PB4, 128))
    lane = lax.broadcasted_iota(jnp.int32, (1, 128), 1)
    mask = lane < C
    vm = jnp.where(mask, pre, -jnp.inf)
    m = jnp.max(vm, axis=1, keepdims=True)
    ex = jnp.where(mask, jnp.exp(pre - m), 0.0)
    lse = jnp.log(jnp.sum(ex, axis=1, keepdims=True)) + m
    out_ref[...] = pre - lse


def _tc_final(p, g, degp, b, w):
    return pl.pallas_call(
        _tc_final_body,
        grid=(GRID,),
        in_specs=[
            pl.BlockSpec((NC, PB4, 128), lambda i: (0, i, 0)),
            pl.BlockSpec((PB4, 128), lambda i: (i, 0)),
            pl.BlockSpec((NC, PB4 // 2, 128), lambda i: (0, i, 0)),
            pl.BlockSpec((1, C), lambda i: (0, 0)),
            pl.BlockSpec((H, C), lambda i: (0, 0)),
        ],
        out_specs=pl.BlockSpec((4 * PB4, 128), lambda i: (i, 0)),
        out_shape=jax.ShapeDtypeStruct((NP, 128), jnp.float32),
    )(p, g, degp, b, w)


def kernel(x, edge_index, glove, W0, b0, W1, b1, W2, b2):
    eidx = edge_index.astype(jnp.int32).reshape(2, EK, K)
    xp = jnp.pad(x, ((0, NP - N), (0, 0))).reshape(NP // 4, 4 * D)

    degp = _sc_degree(eidx)
    degpk = degp.reshape(NC, NP // 8, 128)

    h0 = _tc_first(xp, glove, W0, degpk)
    p0 = _sc_agg_h(h0.reshape(NP, H), eidx)
    h1 = _tc_mid(p0.reshape(NC, NP // 4, 128), h0, degpk,
                 b0.reshape(1, H), W1)
    p1 = _sc_agg_h(h1.reshape(NP, H), eidx)
    g = _tc_mid2(p1.reshape(NC, NP // 4, 128), h1, degpk,
                 b1.reshape(1, H))
    p2 = _sc_agg_h(g.reshape(NP, H), eidx)
    outp = _tc_final(p2.reshape(NC, NP // 4, 128), g, degpk,
                     b2.reshape(1, C), W2)
    return outp[:N, :C]


# speedup vs baseline: 1.0091x; 1.0091x over previous
import functools

import jax
import jax.numpy as jnp
from jax import lax
from jax.experimental import pallas as pl
from jax.experimental.pallas import tpu as pltpu
from jax.experimental.pallas import tpu_sc as plsc

N, E, D, H, C = 10000, 320000, 128, 32, 40
CP = 64

NC, NS = 2, 16
NTILES = NC * NS
EPT = E // NTILES
K = 125
NCHUNK = EPT // K
EK = E // K
NP = 10240
STRIPE = NP // NS
ZROWS = STRIPE // 8

GRID = 5
PB4 = NP // 4 // GRID
PB2 = NP // 2 // GRID

_mesh = plsc.VectorSubcoreMesh(core_axis_name="c", subcore_axis_name="s")


@functools.partial(
    pl.kernel,
    out_type=jax.ShapeDtypeStruct((NC, NP, 16), jnp.float32),
    mesh=_mesh,
    compiler_params=pltpu.CompilerParams(use_tc_tiling_on_sc=False),
    scratch_types=[
        pltpu.VMEM((NCHUNK, K), jnp.int32),
        pltpu.VMEM((K, 16), jnp.float32),
        pltpu.VMEM((ZROWS, 16), jnp.float32),
        pltpu.VMEM_SHARED((NP, 16), jnp.float32),
        pltpu.SemaphoreType.DMA,
    ],
)
def _sc_degree(eidx_hbm, out_hbm, didx, ones, zbuf, acc, dsem):
    cid = lax.axis_index("c")
    sid = lax.axis_index("s")
    tile = cid * NS + sid

    ci = pltpu.async_copy(
        eidx_hbm.at[1, pl.ds(tile * NCHUNK, NCHUNK)], didx, dsem)

    @pl.loop(0, ZROWS)
    def _(i):
        zbuf[i, :] = jnp.zeros((16,), jnp.float32)

    @pl.loop(0, 8)
    def _(r):
        pltpu.sync_copy(zbuf, acc.at[pl.ds(sid * STRIPE + r * ZROWS, ZROWS)])

    @pl.loop(0, K)
    def _(i):
        ones[i, :] = jnp.ones((16,), jnp.float32)

    ci.wait()
    plsc.subcore_barrier()

    @pl.loop(0, 8)
    def _(j):
        pltpu.async_copy(ones, acc.at[didx.at[j]], dsem, add=True)

    @pl.loop(8, NCHUNK)
    def _(j):
        pltpu.make_async_copy(ones, acc.at[didx.at[0]], dsem).wait()
        pltpu.async_copy(ones, acc.at[didx.at[j]], dsem, add=True)

    @pl.loop(0, 8)
    def _(j):
        pltpu.make_async_copy(ones, acc.at[didx.at[0]], dsem).wait()

    plsc.subcore_barrier()
    pltpu.sync_copy(acc.at[pl.ds(sid * STRIPE, STRIPE)],
                    out_hbm.at[cid, pl.ds(sid * STRIPE, STRIPE)])


def _make_sc_aggregate(width):
    nbuf = 8
    @functools.partial(
        pl.kernel,
        out_type=jax.ShapeDtypeStruct((NC, NP, width), jnp.float32),
        mesh=_mesh,
        compiler_params=pltpu.CompilerParams(use_tc_tiling_on_sc=False),
        scratch_types=(
            [pltpu.VMEM((NCHUNK, K), jnp.int32)] * 2
            + [pltpu.VMEM((K, width), jnp.float32)] * nbuf
            + [pltpu.VMEM((ZROWS, width), jnp.float32),
               pltpu.VMEM_SHARED((NP, width), jnp.float32)]
            + [pltpu.SemaphoreType.DMA] * (2 * nbuf + 1)
        ),
    )
    def agg(rows_hbm, eidx_hbm, out_hbm, *scr):
        sidx, didx = scr[0], scr[1]
        bufs = scr[2:2 + nbuf]
        zbuf, acc = scr[2 + nbuf], scr[3 + nbuf]
        gsem = scr[4 + nbuf:4 + 2 * nbuf]
        ssem = scr[4 + 2 * nbuf:4 + 3 * nbuf]
        isem = scr[4 + 3 * nbuf]
        cid = lax.axis_index("c")
        sid = lax.axis_index("s")
        tile = cid * NS + sid

        ca = pltpu.async_copy(
            eidx_hbm.at[0, pl.ds(tile * NCHUNK, NCHUNK)], sidx, isem)
        cb = pltpu.async_copy(
            eidx_hbm.at[1, pl.ds(tile * NCHUNK, NCHUNK)], didx, isem)

        @pl.loop(0, ZROWS)
        def _(i):
            @pl.loop(0, width // 16)
            def _(cc):
                zbuf[i, pl.ds(cc * 16, 16)] = jnp.zeros((16,), jnp.float32)

        @pl.loop(0, 8)
        def _(r):
            pltpu.sync_copy(
                zbuf, acc.at[pl.ds(sid * STRIPE + r * ZROWS, ZROWS)])

        ca.wait()
        cb.wait()

        for u in range(nbuf - 1):
            pltpu.async_copy(rows_hbm.at[sidx.at[u]], bufs[u], gsem[u])

        plsc.subcore_barrier()

        @pl.loop(0, NCHUNK, step=nbuf)
        def _(j):
            for u in range(nbuf):
                c = j + u
                pn = (u + nbuf - 1) % nbuf
                pltpu.make_async_copy(
                    rows_hbm.at[sidx.at[c]], bufs[u], gsem[u]).wait()
                pltpu.async_copy(bufs[u], acc.at[didx.at[c]], ssem[u],
                                 add=True)

                @pl.when(c >= 1)
                def _():
                    pltpu.make_async_copy(
                        bufs[pn], acc.at[didx.at[c - 1]], ssem[pn]).wait()

                @pl.when(c + nbuf - 1 < NCHUNK)
                def _():
                    pltpu.async_copy(
                        rows_hbm.at[sidx.at[c + nbuf - 1]], bufs[pn],
                        gsem[pn])

        pltpu.make_async_copy(
            bufs[(NCHUNK - 1) % nbuf], acc.at[didx.at[NCHUNK - 1]],
            ssem[(NCHUNK - 1) % nbuf]).wait()
        plsc.subcore_barrier()
        pltpu.sync_copy(acc.at[pl.ds(sid * STRIPE, STRIPE)],
                        out_hbm.at[cid, pl.ds(sid * STRIPE, STRIPE)])

    return agg


_sc_agg_h = _make_sc_aggregate(H)


def _lane_shift(w, off, total=128):
    parts = []
    if off:
        parts.append(jnp.zeros((w.shape[0], off), w.dtype))
    parts.append(w)
    rem = total - off - w.shape[1]
    if rem:
        parts.append(jnp.zeros((w.shape[0], rem), w.dtype))
    return jnp.concatenate(parts, axis=1) if len(parts) > 1 else w


def _blockdiag4(w):
    return jnp.concatenate([_lane_shift(w, 32 * a) for a in range(4)], axis=0)


def _expand16(x):
    return jnp.concatenate(
        [x[:, 16 * g:16 * g + 16] for g in (0, 0, 1, 1, 2, 2, 3, 3)], axis=1)


def _dinv4(degp_ref):
    d16 = lax.rsqrt(degp_ref[0] + degp_ref[1] + 1.0)
    return _interleave(_expand16(d16[:, :64]), _expand16(d16[:, 64:]))


def _tc_first_body(xp_ref, glove_ref, w0_ref, degp_ref, out_ref):
    w0p = jnp.dot(glove_ref[...], w0_ref[...],
                  preferred_element_type=jnp.float32)
    wblk = jnp.concatenate(
        [_lane_shift(w0p, 32 * a) for a in range(4)], axis=0)
    h = jnp.dot(xp_ref[...], wblk, preferred_element_type=jnp.float32)
    out_ref[...] = h * _dinv4(degp_ref)


def _tc_first(xp, glove, W0, degp):
    return pl.pallas_call(
        _tc_first_body,
        grid=(GRID,),
        in_specs=[
            pl.BlockSpec((PB4, 4 * D), lambda i: (i, 0)),
            pl.BlockSpec((D, D), lambda i: (0, 0)),
            pl.BlockSpec((D, H), lambda i: (0, 0)),
            pl.BlockSpec((NC, PB4 // 2, 128), lambda i: (0, i, 0)),
        ],
        out_specs=pl.BlockSpec((PB4, 128), lambda i: (i, 0)),
        out_shape=jax.ShapeDtypeStruct((NP // 4, 128), jnp.float32),
    )(xp, glove, W0, degp)


def _pre_act(p_ref, h_ref, b_ref, dinv):
    bt = jnp.concatenate([b_ref[...]] * 4, axis=1)
    pre = dinv * (p_ref[0] + p_ref[1] + h_ref[...]) + bt
    return jnp.maximum(pre, 0.0)


def _tc_mid_body(p_ref, h_ref, degp_ref, b_ref, w_ref, out_ref):
    dinv = _dinv4(degp_ref)
    act = _pre_act(p_ref, h_ref, b_ref, dinv)
    wblk = _blockdiag4(w_ref[...])
    out_ref[...] = jnp.dot(
        act, wblk, preferred_element_type=jnp.float32) * dinv


def _tc_mid(p, hprev, degp, b, w):
    return pl.pallas_call(
        _tc_mid_body,
        grid=(GRID,),
        in_specs=[
            pl.BlockSpec((NC, PB4, 128), lambda i: (0, i, 0)),
            pl.BlockSpec((PB4, 128), lambda i: (i, 0)),
            pl.BlockSpec((NC, PB4 // 2, 128), lambda i: (0, i, 0)),
            pl.BlockSpec((1, H), lambda i: (0, 0)),
            pl.BlockSpec((H, H), lambda i: (0, 0)),
        ],
        out_specs=pl.BlockSpec((PB4, 128), lambda i: (i, 0)),
        out_shape=jax.ShapeDtypeStruct((NP // 4, 128), jnp.float32),
    )(p, hprev, degp, b, w)


def _interleave(even, odd):
    r = even.shape[0]
    st = jnp.concatenate([jnp.reshape(even, (r, 1, 128)),
                          jnp.reshape(odd, (r, 1, 128))], axis=1)
    return jnp.reshape(st, (2 * r, 128))


def _regroup64(d4, lo):
    s = 0 if lo else 64
    a = d4[:, s:s + 32]
    b = d4[:, s + 32:s + 64]
    return jnp.concatenate([a, a, b, b], axis=1)


def _tc_mid2_body(p_ref, h_ref, degp_ref, b_ref, out_ref):
    dinv = _dinv4(degp_ref)
    act = _pre_act(p_ref, h_ref, b_ref, dinv)
    out_ref[...] = act * dinv


def _tc_mid2(p, hprev, degp, b):
    return pl.pallas_call(
        _tc_mid2_body,
        grid=(GRID,),
        in_specs=[
            pl.BlockSpec((NC, PB4, 128), lambda i: (0, i, 0)),
            pl.BlockSpec((PB4, 128), lambda i: (i, 0)),
            pl.BlockSpec((NC, PB4 // 2, 128), lambda i: (0, i, 0)),
            pl.BlockSpec((1, H), lambda i: (0, 0)),
        ],
        out_specs=pl.BlockSpec((PB4, 128), lambda i: (i, 0)),
        out_shape=jax.ShapeDtypeStruct((NP // 4, 128), jnp.float32),
    )(p, hprev, degp, b)


def _tc_final_body(p_ref, g_ref, degp_ref, b_ref, w_ref, out_ref):
    dinv = _dinv4(degp_ref)
    sagg = p_ref[0] + p_ref[1] + g_ref[...]
    w2e = _lane_shift(w_ref[...], 0, 128)
    bt = _lane_shift(b_ref[...], 0, 128)
    rows = []
    for a in range(4):
        sa = sagg[:, 32 * a:32 * a + 32]
        va = jnp.dot(sa, w2e, preferred_element_type=jnp.float32)
        da = dinv[:, 32 * a:32 * a + 1]
        rows.append(jnp.reshape(va * da + bt, (PB4, 1, 128)))
    pre = jnp.reshape(jnp.concatenate(rows, axis=1), (4 * PB4, 128))
    lane = lax.broadcasted_iota(jnp.int32, (1, 128), 1)
    mask = lane < C
    vm = jnp.where(mask, pre, -jnp.inf)
    m = jnp.max(vm, axis=1, keepdims=True)
    ex = jnp.where(mask, jnp.exp(pre - m), 0.0)
    lse = jnp.log(jnp.sum(ex, axis=1, keepdims=True)) + m
    out_ref[...] = pre - lse


def _tc_final(p, g, degp, b, w):
    return pl.pallas_call(
        _tc_final_body,
        grid=(GRID,),
        in_specs=[
            pl.BlockSpec((NC, PB4, 128), lambda i: (0, i, 0)),
            pl.BlockSpec((PB4, 128), lambda i: (i, 0)),
            pl.BlockSpec((NC, PB4 // 2, 128), lambda i: (0, i, 0)),
            pl.BlockSpec((1, C), lambda i: (0, 0)),
            pl.BlockSpec((H, C), lambda i: (0, 0)),
        ],
        out_specs=pl.BlockSpec((4 * PB4, 128), lambda i: (i, 0)),
        out_shape=jax.ShapeDtypeStruct((NP, 128), jnp.float32),
    )(p, g, degp, b, w)


def kernel(x, edge_index, glove, W0, b0, W1, b1, W2, b2):
    eidx = edge_index.astype(jnp.int32).reshape(2, EK, K)
    xp = jnp.pad(x, ((0, NP - N), (0, 0))).reshape(NP // 4, 4 * D)

    degp = _sc_degree(eidx)
    degpk = degp.reshape(NC, NP // 8, 128)

    h0 = _tc_first(xp, glove, W0, degpk)
    p0 = _sc_agg_h(h0.reshape(NP, H), eidx)
    h1 = _tc_mid(p0.reshape(NC, NP // 4, 128), h0, degpk,
                 b0.reshape(1, H), W1)
    p1 = _sc_agg_h(h1.reshape(NP, H), eidx)
    g = _tc_mid2(p1.reshape(NC, NP // 4, 128), h1, degpk,
                 b1.reshape(1, H))
    p2 = _sc_agg_h(g.reshape(NP, H), eidx)
    outp = _tc_final(p2.reshape(NC, NP // 4, 128), g, degpk,
                     b2.reshape(1, C), W2)
    return outp[:N, :C]
